# baseline (speedup 1.0000x reference)
import functools

import jax
import jax.numpy as jnp
from jax import lax
from jax.experimental import pallas as pl
from jax.experimental.pallas import tpu as pltpu
from jax.experimental.pallas import tpu_sc as plsc

B = 1024
L = 200
H = 64
OUT = 100000

NC = 2
NS = 16
NW = NC * NS
LANE = 16
B_PER_W = B // NW

CHUNK_A = 128
CHUNK_B = L - CHUNK_A


def _pool_body(ids_hbm, emb_hbm, out_hbm, idx_a, idx_b, rows_a, rows_b,
               pooled, sem):
    wid = lax.axis_index("s") * NC + lax.axis_index("c")
    base = wid * B_PER_W

    def row_body(r, carry):
        bidx = base + r
        off = pl.multiple_of(bidx * L, 8)
        pltpu.sync_copy(ids_hbm.at[pl.ds(off, CHUNK_A)], idx_a)
        pltpu.sync_copy(ids_hbm.at[pl.ds(off + CHUNK_A, CHUNK_B)], idx_b)
        cp1 = pltpu.async_copy(emb_hbm.at[idx_a], rows_a, sem)
        cp2 = pltpu.async_copy(emb_hbm.at[idx_b], rows_b, sem)
        cp1.wait()
        cp2.wait()

        def tok_a(t, accs):
            return tuple(accs[g] + rows_a[t, pl.ds(g * LANE, LANE)]
                         for g in range(H // LANE))

        def tok_b(t, accs):
            return tuple(accs[g] + rows_b[t, pl.ds(g * LANE, LANE)]
                         for g in range(H // LANE))

        accs = tuple(jnp.zeros((LANE,), jnp.float32) for _ in range(H // LANE))
        accs = lax.fori_loop(0, CHUNK_A, tok_a, accs)
        accs = lax.fori_loop(0, CHUNK_B, tok_b, accs)
        for g in range(H // LANE):
            pooled[r, pl.ds(g * LANE, LANE)] = accs[g]
        return carry

    lax.fori_loop(0, B_PER_W, row_body, 0)
    pltpu.sync_copy(pooled, out_hbm.at[pl.ds(base, B_PER_W)])


def _make_pool():
    mesh = plsc.VectorSubcoreMesh(core_axis_name="c", subcore_axis_name="s",
                                  num_cores=NC, num_subcores=NS)
    return pl.kernel(
        _pool_body,
        out_type=jax.ShapeDtypeStruct((B, H), jnp.float32),
        mesh=mesh,
        scratch_types=[
            pltpu.VMEM((CHUNK_A,), jnp.int32),
            pltpu.VMEM((CHUNK_B,), jnp.int32),
            pltpu.VMEM((CHUNK_A, H), jnp.float32),
            pltpu.VMEM((CHUNK_B, H), jnp.float32),
            pltpu.VMEM((B_PER_W, H), jnp.float32),
            pltpu.SemaphoreType.DMA,
        ],
        compiler_params=pltpu.CompilerParams(use_tc_tiling_on_sc=False),
    )


BO = 1024


def _mm_body(x_ref, w_ref, b_ref, o_ref):
    x = jnp.maximum(x_ref[...] * (1.0 / L), 0.0)
    o_ref[...] = lax.dot_general(
        x, w_ref[...], (((1,), (1,)), ((), ())),
        preferred_element_type=jnp.float32) + b_ref[...]


def _make_mm():
    grid = (pl.cdiv(OUT, BO),)
    return pl.pallas_call(
        _mm_body,
        grid=grid,
        in_specs=[
            pl.BlockSpec((B, H), lambda i: (0, 0)),
            pl.BlockSpec((BO, H), lambda i: (i, 0)),
            pl.BlockSpec((1, BO), lambda i: (0, i)),
        ],
        out_specs=pl.BlockSpec((B, BO), lambda i: (0, i)),
        out_shape=jax.ShapeDtypeStruct((B, OUT), jnp.float32),
    )


@jax.jit
def kernel(input_ids, token_type_ids, attention_mask, emb, W, b):
    ids_flat = input_ids.reshape(-1).astype(jnp.int32)
    pooled = _make_pool()(ids_flat, emb)
    logits = _make_mm()(pooled, W, b.reshape(1, OUT))
    return logits


# speedup vs baseline: 1.3854x; 1.3854x over previous
import functools

import jax
import jax.numpy as jnp
from jax import lax
from jax.experimental import pallas as pl
from jax.experimental.pallas import tpu as pltpu
from jax.experimental.pallas import tpu_sc as plsc

B = 1024
L = 200
H = 64
OUT = 100000

NC = 2
NS = 16
NW = NC * NS
LANE = 16
B_PER_W = B // NW

CHUNK_A = 128
CHUNK_B = L - CHUNK_A


def _pool_body(ids_hbm, emb_hbm, out_hbm, idx_a, idx_b, rows_a, rows_b,
               pooled, sem):
    wid = lax.axis_index("s") * NC + lax.axis_index("c")
    base = wid * B_PER_W

    def row_body(r, carry):
        bidx = base + r
        off = pl.multiple_of(bidx * L, 8)
        pltpu.sync_copy(ids_hbm.at[pl.ds(off, CHUNK_A)], idx_a)
        pltpu.sync_copy(ids_hbm.at[pl.ds(off + CHUNK_A, CHUNK_B)], idx_b)
        cp1 = pltpu.async_copy(emb_hbm.at[idx_a], rows_a, sem)
        cp2 = pltpu.async_copy(emb_hbm.at[idx_b], rows_b, sem)
        cp1.wait()
        cp2.wait()

        def tok_a(t, accs):
            return tuple(accs[g] + rows_a[t, pl.ds(g * LANE, LANE)]
                         for g in range(H // LANE))

        def tok_b(t, accs):
            return tuple(accs[g] + rows_b[t, pl.ds(g * LANE, LANE)]
                         for g in range(H // LANE))

        accs = tuple(jnp.zeros((LANE,), jnp.float32) for _ in range(H // LANE))
        accs = lax.fori_loop(0, CHUNK_A, tok_a, accs)
        accs = lax.fori_loop(0, CHUNK_B, tok_b, accs)
        for g in range(H // LANE):
            pooled[r, pl.ds(g * LANE, LANE)] = accs[g]
        return carry

    lax.fori_loop(0, B_PER_W, row_body, 0)
    pltpu.sync_copy(pooled, out_hbm.at[pl.ds(base, B_PER_W)])


def _make_pool():
    mesh = plsc.VectorSubcoreMesh(core_axis_name="c", subcore_axis_name="s",
                                  num_cores=NC, num_subcores=NS)
    return pl.kernel(
        _pool_body,
        out_type=jax.ShapeDtypeStruct((B, H), jnp.float32),
        mesh=mesh,
        scratch_types=[
            pltpu.VMEM((CHUNK_A,), jnp.int32),
            pltpu.VMEM((CHUNK_B,), jnp.int32),
            pltpu.VMEM((CHUNK_A, H), jnp.float32),
            pltpu.VMEM((CHUNK_B, H), jnp.float32),
            pltpu.VMEM((B_PER_W, H), jnp.float32),
            pltpu.SemaphoreType.DMA,
        ],
        compiler_params=pltpu.CompilerParams(use_tc_tiling_on_sc=False),
    )


BO = 1024


def _mm_body(x_ref, wt_ref, b_ref, o_ref):
    x = jnp.maximum(x_ref[...] * (1.0 / L), 0.0)
    o_ref[...] = lax.dot_general(
        wt_ref[...], x, (((0,), (1,)), ((), ())),
        preferred_element_type=jnp.float32) + b_ref[...]


def _make_mm():
    grid = (pl.cdiv(OUT, BO),)
    return pl.pallas_call(
        _mm_body,
        grid=grid,
        in_specs=[
            pl.BlockSpec((B, H), lambda i: (0, 0)),
            pl.BlockSpec((H, BO), lambda i: (0, i)),
            pl.BlockSpec((BO, 1), lambda i: (i, 0)),
        ],
        out_specs=pl.BlockSpec((BO, B), lambda i: (i, 0)),
        out_shape=jax.ShapeDtypeStruct((OUT, B), jnp.float32),
    )


@jax.jit
def kernel(input_ids, token_type_ids, attention_mask, emb, W, b):
    ids_flat = input_ids.reshape(-1).astype(jnp.int32)
    pooled = _make_pool()(ids_flat, emb)
    logits_t = _make_mm()(pooled, W.T, b.reshape(OUT, 1))
    return logits_t.T
